# Initial kernel scaffold; baseline (speedup 1.0000x reference)
import jax
import jax.numpy as jnp
from jax.experimental import pallas as pl


def kernel(x_enc, y, codebook, W1, b1, W2, b2):
    raise NotImplementedError("write your pallas kernel here")


# speedup vs baseline: 4.5857x; 4.5857x over previous
import jax
import jax.numpy as jnp
from jax.experimental import pallas as pl


def _pq_kernel(z_ref, cb_ref, recon_ref, *, M, K, sub):
    BN = z_ref.shape[0]
    iota = jax.lax.broadcasted_iota(jnp.int32, (BN, K), 1)
    for m in range(M):
        zm = z_ref[:, m * sub:(m + 1) * sub]
        cbm = cb_ref[m]
        dots = jnp.dot(zm, cbm.T, preferred_element_type=jnp.float32)
        z2 = jnp.sum(zm * zm, axis=1, keepdims=True)
        c2 = jnp.sum(cbm * cbm, axis=1)[None, :]
        dist = z2 - 2.0 * dots + c2
        mn = jnp.min(dist, axis=1, keepdims=True)
        idx = jnp.min(jnp.where(dist == mn, iota, K), axis=1)
        oh = (iota == idx[:, None]).astype(jnp.float32)
        recon_ref[:, m * sub:(m + 1) * sub] = jnp.dot(
            oh, cbm, preferred_element_type=jnp.float32)


def _mlp_kernel(flat_ref, w1_ref, b1_ref, w2_ref, b2_ref, out_ref):
    j = pl.program_id(1)
    h = jnp.dot(flat_ref[...], w1_ref[...], preferred_element_type=jnp.float32)
    h = jnp.maximum(h + b1_ref[...], 0.0)
    part = jnp.dot(h, w2_ref[...], preferred_element_type=jnp.float32)

    @pl.when(j == 0)
    def _():
        out_ref[...] = part + b2_ref[...]

    @pl.when(j != 0)
    def _():
        out_ref[...] += part


def _loss_kernel(logits_ref, y_ref, loss_ref):
    l = logits_ref[...]
    B, Tp = l.shape
    mx = jnp.max(l, axis=1, keepdims=True)
    lse = jnp.log(jnp.sum(jnp.exp(l - mx), axis=1, keepdims=True)) + mx
    cols = jax.lax.broadcasted_iota(jnp.int32, (B, Tp), 1)
    oh = (cols == y_ref[...]).astype(jnp.float32)
    ly = jnp.sum(l * oh, axis=1, keepdims=True)
    loss_ref[...] = jnp.mean(lse - ly).reshape(1, 1)


def kernel(x_enc, y, codebook, W1, b1, W2, b2):
    B, C, H, W = x_enc.shape
    M, K, sub = codebook.shape
    N = B * H * W
    d_in = C * H * W
    hidden = W1.shape[1]
    tasks = W2.shape[1]
    Tp = 128

    z = jnp.transpose(x_enc, (0, 2, 3, 1)).reshape(N, C)

    BN = 512
    recon = pl.pallas_call(
        lambda zr, cr, rr: _pq_kernel(zr, cr, rr, M=M, K=K, sub=sub),
        grid=(N // BN,),
        in_specs=[
            pl.BlockSpec((BN, C), lambda i: (i, 0)),
            pl.BlockSpec((M, K, sub), lambda i: (0, 0, 0)),
        ],
        out_specs=pl.BlockSpec((BN, C), lambda i: (i, 0)),
        out_shape=jax.ShapeDtypeStruct((N, C), jnp.float32),
    )(z, codebook)

    flat = recon.reshape(B, H * W, C).transpose(0, 2, 1).reshape(B, d_in)

    W2p = jnp.pad(W2, ((0, 0), (0, Tp - tasks)))
    b2p = jnp.pad(b2, (0, Tp - tasks), constant_values=-1e30).reshape(1, Tp)
    b1r = b1.reshape(1, hidden)

    BB, BH = 256, 512
    logits_p = pl.pallas_call(
        _mlp_kernel,
        grid=(B // BB, hidden // BH),
        in_specs=[
            pl.BlockSpec((BB, d_in), lambda i, j: (i, 0)),
            pl.BlockSpec((d_in, BH), lambda i, j: (0, j)),
            pl.BlockSpec((1, BH), lambda i, j: (0, j)),
            pl.BlockSpec((BH, Tp), lambda i, j: (j, 0)),
            pl.BlockSpec((1, Tp), lambda i, j: (0, 0)),
        ],
        out_specs=pl.BlockSpec((BB, Tp), lambda i, j: (i, 0)),
        out_shape=jax.ShapeDtypeStruct((B, Tp), jnp.float32),
    )(flat, W1, b1r, W2p, b2p)

    y2 = y.astype(jnp.int32).reshape(B, 1)
    loss = pl.pallas_call(
        _loss_kernel,
        grid=(1,),
        in_specs=[
            pl.BlockSpec((B, Tp), lambda i: (0, 0)),
            pl.BlockSpec((B, 1), lambda i: (0, 0)),
        ],
        out_specs=pl.BlockSpec((1, 1), lambda i: (0, 0)),
        out_shape=jax.ShapeDtypeStruct((1, 1), jnp.float32),
    )(logits_p, y2)

    return logits_p[:, :tasks], loss[0, 0]
